# baseline (speedup 1.0000x reference)
import functools

import jax
import jax.numpy as jnp
from jax import lax
from jax.experimental import pallas as pl
from jax.experimental.pallas import tpu as pltpu
from jax.experimental.pallas import tpu_sc as plsc

_NC = 2
_NS = 16
_NW = _NC * _NS
_L = 16


def _pool_body(ap_ref, tgt_ref, pooled_ref, lab_ref):
    x = ap_ref[...]
    pooled_ref[...] = jnp.max(x, axis=1)
    t = tgt_ref[...]
    cls = lax.broadcasted_iota(jnp.int32, t.shape, 1).astype(jnp.float32)
    lab_ref[...] = jnp.sum(t * cls, axis=1, keepdims=True).astype(jnp.int32)


def _final_body(da_ref, dv_ref, out_ref):
    a = da_ref[...]
    v = dv_ref[...]

    def softplus(d):
        return jnp.maximum(d, 0.0) + jnp.log1p(jnp.exp(-jnp.abs(d)))

    s = jnp.sum(softplus(a)) + jnp.sum(softplus(v))
    cnt = jnp.sum((a > -1e29).astype(jnp.float32))
    out_ref[...] = jnp.reshape(s / jnp.maximum(cnt, 1.0), (1, 1))


def _make_sc_pairs(n, bn, c, ppw):
    c2 = 2 * c
    mesh = plsc.VectorSubcoreMesh(core_axis_name="c", subcore_axis_name="s")

    @functools.partial(
        pl.kernel,
        mesh=mesh,
        compiler_params=pltpu.CompilerParams(needs_layout_passes=False),
        out_type=[
            jax.ShapeDtypeStruct((n,), jnp.float32),
            jax.ShapeDtypeStruct((n,), jnp.float32),
        ],
        scratch_types=[
            pltpu.VMEM((bn,), jnp.int32),
            pltpu.VMEM((ppw,), jnp.int32),
            pltpu.VMEM((ppw,), jnp.int32),
            pltpu.VMEM((ppw,), jnp.int32),
            pltpu.VMEM((ppw,), jnp.int32),
            pltpu.VMEM((ppw,), jnp.int32),
            pltpu.VMEM((ppw,), jnp.int32),
            pltpu.VMEM((ppw,), jnp.float32),
            pltpu.VMEM((ppw,), jnp.float32),
            pltpu.VMEM((ppw,), jnp.float32),
            pltpu.VMEM((ppw,), jnp.float32),
            pltpu.VMEM((ppw,), jnp.float32),
            pltpu.VMEM((ppw,), jnp.float32),
            pltpu.VMEM((ppw,), jnp.float32),
            pltpu.SemaphoreType.DMA,
        ],
    )
    def sc_pairs(pooled_hbm, labels_hbm, aidx_hbm, vidx_hbm, da_hbm, dv_hbm,
                 labels_v, aidx_v, vidx_v, ia_v, ivp_v, ian_v, ivn_v,
                 va_v, vvp_v, van_v, vvn_v, pen_v, da_v, dv_v, sem):
        wid = lax.axis_index("s") * _NC + lax.axis_index("c")
        base = wid * ppw
        pltpu.sync_copy(labels_hbm, labels_v)
        pltpu.sync_copy(aidx_hbm.at[pl.ds(base, ppw)], aidx_v)
        pltpu.sync_copy(vidx_hbm.at[pl.ds(base, ppw)], vidx_v)
        for k in range(ppw // _L):
            sl = pl.ds(k * _L, _L)
            ja = aidx_v[sl]
            jv = vidx_v[sl]
            la = plsc.load_gather(labels_v, [ja])
            lv = plsc.load_gather(labels_v, [jv])
            i_vec = lax.iota(jnp.int32, _L) + (base + k * _L)
            ia_v[sl] = ja * c2 + la
            ivp_v[sl] = jv * c2 + (c + lv)
            ian_v[sl] = (i_vec + bn) * c2 + (c + la)
            ivn_v[sl] = (i_vec + bn) * c2 + lv
            pen_v[sl] = jnp.where(la != lv, 0.0, -1e30)
        copies = [
            pltpu.async_copy(pooled_hbm.at[ia_v], va_v, sem),
            pltpu.async_copy(pooled_hbm.at[ivp_v], vvp_v, sem),
            pltpu.async_copy(pooled_hbm.at[ian_v], van_v, sem),
            pltpu.async_copy(pooled_hbm.at[ivn_v], vvn_v, sem),
        ]
        for cp in copies:
            cp.wait()
        for k in range(ppw // _L):
            sl = pl.ds(k * _L, _L)
            da_v[sl] = (van_v[sl] - va_v[sl]) + pen_v[sl]
            dv_v[sl] = (vvn_v[sl] - vvp_v[sl]) + pen_v[sl]
        pltpu.sync_copy(da_v, da_hbm.at[pl.ds(base, ppw)])
        pltpu.sync_copy(dv_v, dv_hbm.at[pl.ds(base, ppw)])

    return sc_pairs


def kernel(all_prob, audio_idx, vis_idx, target):
    total, t_len, _, c = all_prob.shape
    n = audio_idx.shape[0]
    bn = total - n
    c2 = 2 * c

    ap3 = all_prob.reshape(total, t_len, c2)
    audio_idx = audio_idx.astype(jnp.int32)
    vis_idx = vis_idx.astype(jnp.int32)

    grid = 64
    r = total // grid
    rt = bn // grid
    pooled, labels2d = pl.pallas_call(
        _pool_body,
        grid=(grid,),
        in_specs=[
            pl.BlockSpec((r, t_len, c2), lambda i: (i, 0, 0)),
            pl.BlockSpec((rt, c), lambda i: (i, 0)),
        ],
        out_specs=[
            pl.BlockSpec((r, c2), lambda i: (i, 0)),
            pl.BlockSpec((rt, 1), lambda i: (i, 0)),
        ],
        out_shape=[
            jax.ShapeDtypeStruct((total, c2), jnp.float32),
            jax.ShapeDtypeStruct((bn, 1), jnp.int32),
        ],
    )(ap3, target)

    ppw = n // _NW
    sc_pairs = _make_sc_pairs(n, bn, c, ppw)
    da, dv = sc_pairs(pooled.reshape(total * c2), labels2d.reshape(bn),
                      audio_idx, vis_idx)

    rows = n // 128
    out = pl.pallas_call(
        _final_body,
        out_shape=jax.ShapeDtypeStruct((1, 1), jnp.float32),
    )(da.reshape(rows, 128), dv.reshape(rows, 128))
    return out[0, 0]


# speedup vs baseline: 6.3252x; 6.3252x over previous
import functools

import jax
import jax.numpy as jnp
from jax import lax
from jax.experimental import pallas as pl
from jax.experimental.pallas import tpu as pltpu
from jax.experimental.pallas import tpu_sc as plsc

_NC = 2
_NS = 16
_NW = _NC * _NS
_L = 16


def _pool_body(ap_ref, pooled_ref):
    pooled_ref[...] = jnp.max(ap_ref[...], axis=0)


def _labels_body(tgt_ref, lab_ref):
    t = tgt_ref[...]
    cls = lax.broadcasted_iota(jnp.int32, t.shape, 0).astype(jnp.float32)
    lab_ref[...] = jnp.sum(t * cls, axis=0, keepdims=True).astype(jnp.int32)


def _final_body(da_ref, dv_ref, out_ref):
    a = da_ref[...]
    v = dv_ref[...]

    def softplus(d):
        return jnp.maximum(d, 0.0) + jnp.log1p(jnp.exp(-jnp.abs(d)))

    s = jnp.sum(softplus(a)) + jnp.sum(softplus(v))
    cnt = jnp.sum((a > -1e29).astype(jnp.float32))
    out_ref[...] = jnp.reshape(s / jnp.maximum(cnt, 1.0), (1, 1))


def _make_sc_pairs(n, bn, c, ppw):
    mesh = plsc.VectorSubcoreMesh(core_axis_name="c", subcore_axis_name="s")

    @functools.partial(
        pl.kernel,
        mesh=mesh,
        compiler_params=pltpu.CompilerParams(needs_layout_passes=False),
        out_type=[
            jax.ShapeDtypeStruct((n,), jnp.float32),
            jax.ShapeDtypeStruct((n,), jnp.float32),
        ],
        scratch_types=[
            pltpu.VMEM((bn,), jnp.int32),
            pltpu.VMEM((ppw,), jnp.int32),
            pltpu.VMEM((ppw,), jnp.int32),
            pltpu.VMEM((ppw,), jnp.int32),
            pltpu.VMEM((ppw,), jnp.int32),
            pltpu.VMEM((ppw,), jnp.int32),
            pltpu.VMEM((ppw,), jnp.int32),
            pltpu.VMEM((ppw,), jnp.float32),
            pltpu.VMEM((ppw,), jnp.float32),
            pltpu.VMEM((ppw,), jnp.float32),
            pltpu.VMEM((ppw,), jnp.float32),
            pltpu.VMEM((ppw,), jnp.float32),
            pltpu.VMEM((ppw,), jnp.float32),
            pltpu.VMEM((ppw,), jnp.float32),
            pltpu.SemaphoreType.DMA,
        ],
    )
    def sc_pairs(pooled_hbm, labels_hbm, aidx_hbm, vidx_hbm, da_hbm, dv_hbm,
                 labels_v, aidx_v, vidx_v, ia_v, ivp_v, ian_v, ivn_v,
                 va_v, vvp_v, van_v, vvn_v, pen_v, da_v, dv_v, sem):
        cstride = (bn + n) // 128 * 256
        wid = lax.axis_index("s") * _NC + lax.axis_index("c")
        base = wid * ppw
        pltpu.sync_copy(labels_hbm, labels_v)
        pltpu.sync_copy(aidx_hbm.at[pl.ds(base, ppw)], aidx_v)
        pltpu.sync_copy(vidx_hbm.at[pl.ds(base, ppw)], vidx_v)
        for k in range(ppw // _L):
            sl = pl.ds(k * _L, _L)
            ja = aidx_v[sl]
            jv = vidx_v[sl]
            la = plsc.load_gather(labels_v, [ja])
            lv = plsc.load_gather(labels_v, [jv])
            r = lax.iota(jnp.int32, _L) + (base + k * _L + bn)
            ja_off = lax.shift_right_logical(ja, 7) * 256 + (ja & 127)
            jv_off = lax.shift_right_logical(jv, 7) * 256 + (jv & 127)
            r_off = lax.shift_right_logical(r, 7) * 256 + (r & 127)
            ia_v[sl] = la * cstride + ja_off
            ivp_v[sl] = lv * cstride + jv_off + 128
            ian_v[sl] = la * cstride + r_off + 128
            ivn_v[sl] = lv * cstride + r_off
            pen_v[sl] = jnp.where(la != lv, 0.0, -1e30)
        copies = [
            pltpu.async_copy(pooled_hbm.at[ia_v], va_v, sem),
            pltpu.async_copy(pooled_hbm.at[ivp_v], vvp_v, sem),
            pltpu.async_copy(pooled_hbm.at[ian_v], van_v, sem),
            pltpu.async_copy(pooled_hbm.at[ivn_v], vvn_v, sem),
        ]
        for cp in copies:
            cp.wait()
        for k in range(ppw // _L):
            sl = pl.ds(k * _L, _L)
            da_v[sl] = (van_v[sl] - va_v[sl]) + pen_v[sl]
            dv_v[sl] = (vvn_v[sl] - vvp_v[sl]) + pen_v[sl]
        pltpu.sync_copy(da_v, da_hbm.at[pl.ds(base, ppw)])
        pltpu.sync_copy(dv_v, dv_hbm.at[pl.ds(base, ppw)])

    return sc_pairs


def kernel(all_prob, audio_idx, vis_idx, target):
    total, t_len, _, c = all_prob.shape
    n = audio_idx.shape[0]
    bn = total - n
    rb = total // 128

    audio_idx = audio_idx.astype(jnp.int32)
    vis_idx = vis_idx.astype(jnp.int32)

    ap_v = (all_prob.reshape(rb, 128, t_len, 2, c)
            .transpose(2, 4, 0, 3, 1)
            .reshape(t_len, c, rb * 2, 128))

    pooled = pl.pallas_call(
        _pool_body,
        grid=(c,),
        in_specs=[pl.BlockSpec((t_len, 1, rb * 2, 128), lambda i: (0, i, 0, 0))],
        out_specs=pl.BlockSpec((1, rb * 2, 128), lambda i: (i, 0, 0)),
        out_shape=jax.ShapeDtypeStruct((c, rb * 2, 128), jnp.float32),
    )(ap_v)

    labels2d = pl.pallas_call(
        _labels_body,
        out_shape=jax.ShapeDtypeStruct((1, bn), jnp.int32),
    )(target.T)

    ppw = n // _NW
    sc_pairs = _make_sc_pairs(n, bn, c, ppw)
    da, dv = sc_pairs(pooled.reshape(c * rb * 256), labels2d.reshape(bn),
                      audio_idx, vis_idx)

    rows = n // 128
    out = pl.pallas_call(
        _final_body,
        out_shape=jax.ShapeDtypeStruct((1, 1), jnp.float32),
    )(da.reshape(rows, 128), dv.reshape(rows, 128))
    return out[0, 0]


# speedup vs baseline: 6.3643x; 1.0062x over previous
import functools

import jax
import jax.numpy as jnp
from jax import lax
from jax.experimental import pallas as pl
from jax.experimental.pallas import tpu as pltpu
from jax.experimental.pallas import tpu_sc as plsc

_NC = 2
_NS = 16
_NW = _NC * _NS
_L = 16


def _pool_body(ap_ref, tgt_ref, pooled_ref, lab_ref):
    i = pl.program_id(0)
    pooled_ref[...] = jnp.max(ap_ref[...], axis=0)
    contrib = tgt_ref[...].astype(jnp.int32) * i

    @pl.when(i == 0)
    def _():
        lab_ref[...] = jnp.zeros_like(lab_ref)

    lab_ref[...] += contrib


def _final_body(da_ref, dv_ref, out_ref):
    a = da_ref[...]
    v = dv_ref[...]

    def softplus(d):
        return jnp.maximum(d, 0.0) + jnp.log1p(jnp.exp(-jnp.abs(d)))

    s = jnp.sum(softplus(a)) + jnp.sum(softplus(v))
    cnt = jnp.sum((a > -1e29).astype(jnp.float32))
    out_ref[...] = jnp.reshape(s / jnp.maximum(cnt, 1.0), (1, 1))


def _make_sc_pairs(n, bn, c, ppw):
    mesh = plsc.VectorSubcoreMesh(core_axis_name="c", subcore_axis_name="s")

    @functools.partial(
        pl.kernel,
        mesh=mesh,
        compiler_params=pltpu.CompilerParams(needs_layout_passes=False),
        out_type=[
            jax.ShapeDtypeStruct((n,), jnp.float32),
            jax.ShapeDtypeStruct((n,), jnp.float32),
        ],
        scratch_types=[
            pltpu.VMEM((ppw,), jnp.int32),
            pltpu.VMEM((ppw,), jnp.int32),
            pltpu.VMEM((ppw,), jnp.int32),
            pltpu.VMEM((ppw,), jnp.int32),
            pltpu.VMEM((ppw,), jnp.int32),
            pltpu.VMEM((ppw,), jnp.int32),
            pltpu.VMEM((ppw,), jnp.int32),
            pltpu.VMEM((ppw,), jnp.int32),
            pltpu.VMEM((ppw,), jnp.float32),
            pltpu.VMEM((ppw,), jnp.float32),
            pltpu.VMEM((ppw,), jnp.float32),
            pltpu.VMEM((ppw,), jnp.float32),
            pltpu.VMEM((ppw,), jnp.float32),
            pltpu.VMEM((ppw,), jnp.float32),
            pltpu.VMEM((ppw,), jnp.float32),
            pltpu.SemaphoreType.DMA,
        ],
    )
    def sc_pairs(pooled_hbm, labels_hbm, aidx_hbm, vidx_hbm, da_hbm, dv_hbm,
                 aidx_v, vidx_v, la_v, lv_v, ia_v, ivp_v, ian_v, ivn_v,
                 va_v, vvp_v, van_v, vvn_v, pen_v, da_v, dv_v, sem):
        cstride = (bn + n) // 128 * 256
        wid = lax.axis_index("s") * _NC + lax.axis_index("c")
        base = wid * ppw
        pltpu.sync_copy(aidx_hbm.at[pl.ds(base, ppw)], aidx_v)
        pltpu.sync_copy(vidx_hbm.at[pl.ds(base, ppw)], vidx_v)
        lab_copies = [
            pltpu.async_copy(labels_hbm.at[aidx_v], la_v, sem),
            pltpu.async_copy(labels_hbm.at[vidx_v], lv_v, sem),
        ]
        for cp in lab_copies:
            cp.wait()
        for k in range(ppw // _L):
            sl = pl.ds(k * _L, _L)
            ja = aidx_v[sl]
            jv = vidx_v[sl]
            la = la_v[sl]
            lv = lv_v[sl]
            r = lax.iota(jnp.int32, _L) + (base + k * _L + bn)
            ja_off = lax.shift_right_logical(ja, 7) * 256 + (ja & 127)
            jv_off = lax.shift_right_logical(jv, 7) * 256 + (jv & 127)
            r_off = lax.shift_right_logical(r, 7) * 256 + (r & 127)
            ia_v[sl] = la * cstride + ja_off
            ivp_v[sl] = lv * cstride + jv_off + 128
            ian_v[sl] = la * cstride + r_off + 128
            ivn_v[sl] = lv * cstride + r_off
            pen_v[sl] = jnp.where(la != lv, 0.0, -1e30)
        copies = [
            pltpu.async_copy(pooled_hbm.at[ia_v], va_v, sem),
            pltpu.async_copy(pooled_hbm.at[ivp_v], vvp_v, sem),
            pltpu.async_copy(pooled_hbm.at[ian_v], van_v, sem),
            pltpu.async_copy(pooled_hbm.at[ivn_v], vvn_v, sem),
        ]
        for cp in copies:
            cp.wait()
        for k in range(ppw // _L):
            sl = pl.ds(k * _L, _L)
            da_v[sl] = (van_v[sl] - va_v[sl]) + pen_v[sl]
            dv_v[sl] = (vvn_v[sl] - vvp_v[sl]) + pen_v[sl]
        pltpu.sync_copy(da_v, da_hbm.at[pl.ds(base, ppw)])
        pltpu.sync_copy(dv_v, dv_hbm.at[pl.ds(base, ppw)])

    return sc_pairs


def kernel(all_prob, audio_idx, vis_idx, target):
    total, t_len, _, c = all_prob.shape
    n = audio_idx.shape[0]
    bn = total - n
    rb = total // 128

    audio_idx = audio_idx.astype(jnp.int32)
    vis_idx = vis_idx.astype(jnp.int32)

    ap_v = (all_prob.reshape(rb, 128, t_len, 2, c)
            .transpose(2, 4, 0, 3, 1)
            .reshape(t_len, c, rb * 2, 128))

    pooled, labels2d = pl.pallas_call(
        _pool_body,
        grid=(c,),
        in_specs=[
            pl.BlockSpec((t_len, 1, rb * 2, 128), lambda i: (0, i, 0, 0)),
            pl.BlockSpec((1, 1, bn), lambda i: (i, 0, 0)),
        ],
        out_specs=[
            pl.BlockSpec((1, rb * 2, 128), lambda i: (i, 0, 0)),
            pl.BlockSpec((1, 1, bn), lambda i: (0, 0, 0)),
        ],
        out_shape=[
            jax.ShapeDtypeStruct((c, rb * 2, 128), jnp.float32),
            jax.ShapeDtypeStruct((1, 1, bn), jnp.int32),
        ],
    )(ap_v, target.T.reshape(c, 1, bn))

    ppw = n // _NW
    sc_pairs = _make_sc_pairs(n, bn, c, ppw)
    da, dv = sc_pairs(pooled.reshape(c * rb * 256), labels2d.reshape(bn),
                      audio_idx, vis_idx)

    rows = n // 128
    out = pl.pallas_call(
        _final_body,
        out_shape=jax.ShapeDtypeStruct((1, 1), jnp.float32),
    )(da.reshape(rows, 128), dv.reshape(rows, 128))
    return out[0, 0]


# speedup vs baseline: 6.6788x; 1.0494x over previous
import functools

import jax
import jax.numpy as jnp
from jax import lax
from jax.experimental import pallas as pl
from jax.experimental.pallas import tpu as pltpu
from jax.experimental.pallas import tpu_sc as plsc

_NC = 2
_NS = 16
_NW = _NC * _NS
_L = 16


def _pool_body(ap_ref, tgt_ref, pooled_ref, lab_ref):
    i = pl.program_id(0)
    pooled_ref[...] = jnp.max(ap_ref[...], axis=0)
    contrib = tgt_ref[pl.ds(i, 1), :].astype(jnp.int32) * i

    @pl.when(i == 0)
    def _():
        lab_ref[...] = jnp.zeros_like(lab_ref)

    lab_ref[...] += contrib


def _final_body(da_ref, dv_ref, out_ref):
    a = da_ref[...]
    v = dv_ref[...]

    def softplus(d):
        return jnp.maximum(d, 0.0) + jnp.log1p(jnp.exp(-jnp.abs(d)))

    s = jnp.sum(softplus(a)) + jnp.sum(softplus(v))
    cnt = jnp.sum((a > -1e29).astype(jnp.float32))
    out_ref[...] = jnp.reshape(s / jnp.maximum(cnt, 1.0), (1, 1))


def _make_sc_pairs(n, bn, c, ppw):
    mesh = plsc.VectorSubcoreMesh(core_axis_name="c", subcore_axis_name="s")

    @functools.partial(
        pl.kernel,
        mesh=mesh,
        compiler_params=pltpu.CompilerParams(needs_layout_passes=False),
        out_type=[
            jax.ShapeDtypeStruct((n,), jnp.float32),
            jax.ShapeDtypeStruct((n,), jnp.float32),
        ],
        scratch_types=[
            pltpu.VMEM((ppw,), jnp.int32),
            pltpu.VMEM((ppw,), jnp.int32),
            pltpu.VMEM((ppw,), jnp.int32),
            pltpu.VMEM((ppw,), jnp.int32),
            pltpu.VMEM((ppw,), jnp.int32),
            pltpu.VMEM((ppw,), jnp.int32),
            pltpu.VMEM((ppw,), jnp.int32),
            pltpu.VMEM((ppw,), jnp.int32),
            pltpu.VMEM((ppw,), jnp.float32),
            pltpu.VMEM((ppw,), jnp.float32),
            pltpu.VMEM((ppw,), jnp.float32),
            pltpu.VMEM((ppw,), jnp.float32),
            pltpu.VMEM((ppw,), jnp.float32),
            pltpu.VMEM((ppw,), jnp.float32),
            pltpu.VMEM((ppw,), jnp.float32),
            pltpu.SemaphoreType.DMA,
        ],
    )
    def sc_pairs(pooled_hbm, labels_hbm, aidx_hbm, vidx_hbm, da_hbm, dv_hbm,
                 aidx_v, vidx_v, la_v, lv_v, ia_v, ivp_v, ian_v, ivn_v,
                 va_v, vvp_v, van_v, vvn_v, pen_v, da_v, dv_v, sem):
        cstride = (bn + n) // 128 * 256
        wid = lax.axis_index("s") * _NC + lax.axis_index("c")
        base = wid * ppw
        idx_copies = [
            pltpu.async_copy(aidx_hbm.at[pl.ds(base, ppw)], aidx_v, sem),
            pltpu.async_copy(vidx_hbm.at[pl.ds(base, ppw)], vidx_v, sem),
        ]
        for cp in idx_copies:
            cp.wait()
        lab_copies = [
            pltpu.async_copy(labels_hbm.at[aidx_v], la_v, sem),
            pltpu.async_copy(labels_hbm.at[vidx_v], lv_v, sem),
        ]
        for cp in lab_copies:
            cp.wait()

        def build(k, _):
            sl = pl.ds(k * _L, _L)
            ja = aidx_v[sl]
            jv = vidx_v[sl]
            la = la_v[sl]
            lv = lv_v[sl]
            r = lax.iota(jnp.int32, _L) + (base + k * _L + bn)
            ja_off = lax.shift_right_logical(ja, 7) * 256 + (ja & 127)
            jv_off = lax.shift_right_logical(jv, 7) * 256 + (jv & 127)
            r_off = lax.shift_right_logical(r, 7) * 256 + (r & 127)
            ia_v[sl] = la * cstride + ja_off
            ivp_v[sl] = lv * cstride + jv_off + 128
            ian_v[sl] = la * cstride + r_off + 128
            ivn_v[sl] = lv * cstride + r_off
            pen_v[sl] = jnp.where(la != lv, 0.0, -1e30)
            return 0

        lax.fori_loop(0, ppw // _L, build, 0)
        copies = [
            pltpu.async_copy(pooled_hbm.at[ia_v], va_v, sem),
            pltpu.async_copy(pooled_hbm.at[ivp_v], vvp_v, sem),
            pltpu.async_copy(pooled_hbm.at[ian_v], van_v, sem),
            pltpu.async_copy(pooled_hbm.at[ivn_v], vvn_v, sem),
        ]
        for cp in copies:
            cp.wait()

        def diff(k, _):
            sl = pl.ds(k * _L, _L)
            da_v[sl] = (van_v[sl] - va_v[sl]) + pen_v[sl]
            dv_v[sl] = (vvn_v[sl] - vvp_v[sl]) + pen_v[sl]
            return 0

        lax.fori_loop(0, ppw // _L, diff, 0)
        out_copies = [
            pltpu.async_copy(da_v, da_hbm.at[pl.ds(base, ppw)], sem),
            pltpu.async_copy(dv_v, dv_hbm.at[pl.ds(base, ppw)], sem),
        ]
        for cp in out_copies:
            cp.wait()

    return sc_pairs


def kernel(all_prob, audio_idx, vis_idx, target):
    total, t_len, _, c = all_prob.shape
    n = audio_idx.shape[0]
    bn = total - n
    rb = total // 128

    audio_idx = audio_idx.astype(jnp.int32)
    vis_idx = vis_idx.astype(jnp.int32)

    ap_v = (all_prob.reshape(rb, 128, t_len, 2, c)
            .transpose(2, 4, 0, 3, 1)
            .reshape(t_len, c, rb * 2, 128))

    pooled, labels2d = pl.pallas_call(
        _pool_body,
        grid=(c,),
        in_specs=[
            pl.BlockSpec((t_len, 1, rb * 2, 128), lambda i: (0, i, 0, 0)),
            pl.BlockSpec((c, bn), lambda i: (0, 0)),
        ],
        out_specs=[
            pl.BlockSpec((1, rb * 2, 128), lambda i: (i, 0, 0)),
            pl.BlockSpec((1, bn), lambda i: (0, 0)),
        ],
        out_shape=[
            jax.ShapeDtypeStruct((c, rb * 2, 128), jnp.float32),
            jax.ShapeDtypeStruct((1, bn), jnp.int32),
        ],
    )(ap_v, target.T)

    ppw = n // _NW
    sc_pairs = _make_sc_pairs(n, bn, c, ppw)
    da, dv = sc_pairs(pooled.reshape(c * rb * 256), labels2d.reshape(bn),
                      audio_idx, vis_idx)

    rows = n // 128
    out = pl.pallas_call(
        _final_body,
        out_shape=jax.ShapeDtypeStruct((1, 1), jnp.float32),
    )(da.reshape(rows, 128), dv.reshape(rows, 128))
    return out[0, 0]


# speedup vs baseline: 7.0845x; 1.0607x over previous
import functools

import jax
import jax.numpy as jnp
from jax import lax
from jax.experimental import pallas as pl
from jax.experimental.pallas import tpu as pltpu
from jax.experimental.pallas import tpu_sc as plsc

_NC = 2
_NS = 16
_NW = _NC * _NS
_L = 16


def _pool_body(cb, ap_ref, tgt_ref, pooled_ref, lab_ref):
    i = pl.program_id(0)
    pooled_ref[...] = jnp.max(ap_ref[...], axis=0)

    @pl.when(i == 0)
    def _():
        t = tgt_ref[...]
        cls = lax.broadcasted_iota(jnp.int32, t.shape, 0).astype(jnp.float32)
        lab_ref[...] = jnp.sum(t * cls, axis=0, keepdims=True).astype(jnp.int32)


def _final_body(da_ref, dv_ref, out_ref):
    a = da_ref[...]
    v = dv_ref[...]

    def softplus(d):
        return jnp.maximum(d, 0.0) + jnp.log1p(jnp.exp(-jnp.abs(d)))

    s = jnp.sum(softplus(a)) + jnp.sum(softplus(v))
    cnt = jnp.sum((a > -1e29).astype(jnp.float32))
    out_ref[...] = jnp.reshape(s / jnp.maximum(cnt, 1.0), (1, 1))


def _make_sc_pairs(n, bn, c, ppw):
    mesh = plsc.VectorSubcoreMesh(core_axis_name="c", subcore_axis_name="s")

    @functools.partial(
        pl.kernel,
        mesh=mesh,
        compiler_params=pltpu.CompilerParams(needs_layout_passes=False),
        out_type=[
            jax.ShapeDtypeStruct((n,), jnp.float32),
            jax.ShapeDtypeStruct((n,), jnp.float32),
        ],
        scratch_types=[
            pltpu.VMEM((ppw,), jnp.int32),
            pltpu.VMEM((ppw,), jnp.int32),
            pltpu.VMEM((ppw,), jnp.int32),
            pltpu.VMEM((ppw,), jnp.int32),
            pltpu.VMEM((ppw,), jnp.int32),
            pltpu.VMEM((ppw,), jnp.int32),
            pltpu.VMEM((ppw,), jnp.int32),
            pltpu.VMEM((ppw,), jnp.int32),
            pltpu.VMEM((ppw,), jnp.float32),
            pltpu.VMEM((ppw,), jnp.float32),
            pltpu.VMEM((ppw,), jnp.float32),
            pltpu.VMEM((ppw,), jnp.float32),
            pltpu.VMEM((ppw,), jnp.float32),
            pltpu.VMEM((ppw,), jnp.float32),
            pltpu.VMEM((ppw,), jnp.float32),
            pltpu.SemaphoreType.DMA,
        ],
    )
    def sc_pairs(pooled_hbm, labels_hbm, aidx_hbm, vidx_hbm, da_hbm, dv_hbm,
                 aidx_v, vidx_v, la_v, lv_v, ia_v, ivp_v, ian_v, ivn_v,
                 va_v, vvp_v, van_v, vvn_v, pen_v, da_v, dv_v, sem):
        cstride = (bn + n) // 128 * 256
        wid = lax.axis_index("s") * _NC + lax.axis_index("c")
        base = wid * ppw
        idx_copies = [
            pltpu.async_copy(aidx_hbm.at[pl.ds(base, ppw)], aidx_v, sem),
            pltpu.async_copy(vidx_hbm.at[pl.ds(base, ppw)], vidx_v, sem),
        ]
        for cp in idx_copies:
            cp.wait()
        lab_copies = [
            pltpu.async_copy(labels_hbm.at[aidx_v], la_v, sem),
            pltpu.async_copy(labels_hbm.at[vidx_v], lv_v, sem),
        ]
        for cp in lab_copies:
            cp.wait()

        def build(k, _):
            sl = pl.ds(k * _L, _L)
            ja = aidx_v[sl]
            jv = vidx_v[sl]
            la = la_v[sl]
            lv = lv_v[sl]
            r = lax.iota(jnp.int32, _L) + (base + k * _L + bn)
            ja_off = lax.shift_right_logical(ja, 7) * 256 + (ja & 127)
            jv_off = lax.shift_right_logical(jv, 7) * 256 + (jv & 127)
            r_off = lax.shift_right_logical(r, 7) * 256 + (r & 127)
            ia_v[sl] = la * cstride + ja_off
            ivp_v[sl] = lv * cstride + jv_off + 128
            ian_v[sl] = la * cstride + r_off + 128
            ivn_v[sl] = lv * cstride + r_off
            pen_v[sl] = jnp.where(la != lv, 0.0, -1e30)
            return 0

        lax.fori_loop(0, ppw // _L, build, 0)
        copies = [
            pltpu.async_copy(pooled_hbm.at[ia_v], va_v, sem),
            pltpu.async_copy(pooled_hbm.at[ivp_v], vvp_v, sem),
            pltpu.async_copy(pooled_hbm.at[ian_v], van_v, sem),
            pltpu.async_copy(pooled_hbm.at[ivn_v], vvn_v, sem),
        ]
        for cp in copies:
            cp.wait()

        def diff(k, _):
            sl = pl.ds(k * _L, _L)
            da_v[sl] = (van_v[sl] - va_v[sl]) + pen_v[sl]
            dv_v[sl] = (vvn_v[sl] - vvp_v[sl]) + pen_v[sl]
            return 0

        lax.fori_loop(0, ppw // _L, diff, 0)
        out_copies = [
            pltpu.async_copy(da_v, da_hbm.at[pl.ds(base, ppw)], sem),
            pltpu.async_copy(dv_v, dv_hbm.at[pl.ds(base, ppw)], sem),
        ]
        for cp in out_copies:
            cp.wait()

    return sc_pairs


def kernel(all_prob, audio_idx, vis_idx, target):
    total, t_len, _, c = all_prob.shape
    n = audio_idx.shape[0]
    bn = total - n
    rb = total // 128

    audio_idx = audio_idx.astype(jnp.int32)
    vis_idx = vis_idx.astype(jnp.int32)

    ap_v = (all_prob.reshape(rb, 128, t_len, 2, c)
            .transpose(2, 4, 0, 3, 1)
            .reshape(t_len, c, rb * 2, 128))

    cb = 5
    pooled, labels2d = pl.pallas_call(
        functools.partial(_pool_body, cb),
        grid=(c // cb,),
        in_specs=[
            pl.BlockSpec((t_len, cb, rb * 2, 128), lambda i: (0, i, 0, 0)),
            pl.BlockSpec((c, bn), lambda i: (0, 0)),
        ],
        out_specs=[
            pl.BlockSpec((cb, rb * 2, 128), lambda i: (i, 0, 0)),
            pl.BlockSpec((1, bn), lambda i: (0, 0)),
        ],
        out_shape=[
            jax.ShapeDtypeStruct((c, rb * 2, 128), jnp.float32),
            jax.ShapeDtypeStruct((1, bn), jnp.int32),
        ],
    )(ap_v, target.T)

    ppw = n // _NW
    sc_pairs = _make_sc_pairs(n, bn, c, ppw)
    da, dv = sc_pairs(pooled.reshape(c * rb * 256), labels2d.reshape(bn),
                      audio_idx, vis_idx)

    rows = n // 128
    out = pl.pallas_call(
        _final_body,
        out_shape=jax.ShapeDtypeStruct((1, 1), jnp.float32),
    )(da.reshape(rows, 128), dv.reshape(rows, 128))
    return out[0, 0]


# speedup vs baseline: 7.1211x; 1.0052x over previous
import functools

import jax
import jax.numpy as jnp
from jax import lax
from jax.experimental import pallas as pl
from jax.experimental.pallas import tpu as pltpu
from jax.experimental.pallas import tpu_sc as plsc

_NC = 2
_NS = 16
_NW = _NC * _NS
_L = 16


def _pool_body(cb, ap_ref, tgt_ref, pooled_ref, lab_ref):
    i = pl.program_id(0)
    pooled_ref[...] = jnp.max(ap_ref[...], axis=0)

    @pl.when(i == 0)
    def _():
        t = tgt_ref[...]
        cls = lax.broadcasted_iota(jnp.int32, t.shape, 0).astype(jnp.float32)
        lab_ref[...] = jnp.sum(t * cls, axis=0, keepdims=True).astype(jnp.int32)


def _final_body(d_ref, out_ref):
    d = d_ref[...]

    def softplus(x):
        return jnp.maximum(x, 0.0) + jnp.log1p(jnp.exp(-jnp.abs(x)))

    s = jnp.sum(softplus(d))
    cnt = 0.5 * jnp.sum((d > -1e29).astype(jnp.float32))
    out_ref[...] = jnp.reshape(s / jnp.maximum(cnt, 1.0), (1, 1))


def _make_sc_pairs(n, bn, c, ppw):
    mesh = plsc.VectorSubcoreMesh(core_axis_name="c", subcore_axis_name="s")

    @functools.partial(
        pl.kernel,
        mesh=mesh,
        compiler_params=pltpu.CompilerParams(needs_layout_passes=False),
        out_type=jax.ShapeDtypeStruct((2 * n,), jnp.float32),
        scratch_types=[
            pltpu.VMEM((ppw,), jnp.int32),
            pltpu.VMEM((ppw,), jnp.int32),
            pltpu.VMEM((ppw,), jnp.int32),
            pltpu.VMEM((ppw,), jnp.int32),
            pltpu.VMEM((ppw,), jnp.int32),
            pltpu.VMEM((ppw,), jnp.int32),
            pltpu.VMEM((ppw,), jnp.int32),
            pltpu.VMEM((ppw,), jnp.int32),
            pltpu.VMEM((ppw,), jnp.float32),
            pltpu.VMEM((ppw,), jnp.float32),
            pltpu.VMEM((ppw,), jnp.float32),
            pltpu.VMEM((ppw,), jnp.float32),
            pltpu.VMEM((ppw,), jnp.float32),
            pltpu.VMEM((ppw,), jnp.float32),
            pltpu.VMEM((ppw,), jnp.float32),
            pltpu.SemaphoreType.DMA,
            pltpu.SemaphoreType.DMA,
        ],
    )
    def sc_pairs(pooled_hbm, labels_hbm, aidx_hbm, vidx_hbm, d_hbm,
                 aidx_v, vidx_v, la_v, lv_v, ia_v, ivp_v, ian_v, ivn_v,
                 va_v, vvp_v, van_v, vvn_v, pen_v, da_v, dv_v, sem, sem2):
        cstride = (bn + n) // 128 * 256
        wid = lax.axis_index("s") * _NC + lax.axis_index("c")
        base = wid * ppw
        half = ppw // 2
        idx_copies = [
            pltpu.async_copy(aidx_hbm.at[pl.ds(base, ppw)], aidx_v, sem),
            pltpu.async_copy(vidx_hbm.at[pl.ds(base, ppw)], vidx_v, sem),
        ]
        for cp in idx_copies:
            cp.wait()
        lab_copies = [
            pltpu.async_copy(labels_hbm.at[aidx_v], la_v, sem),
            pltpu.async_copy(labels_hbm.at[vidx_v], lv_v, sem),
        ]
        for cp in lab_copies:
            cp.wait()

        def build(k, _):
            sl = pl.ds(k * _L, _L)
            ja = aidx_v[sl]
            jv = vidx_v[sl]
            la = la_v[sl]
            lv = lv_v[sl]
            r = lax.iota(jnp.int32, _L) + (base + k * _L + bn)
            ja_off = lax.shift_right_logical(ja, 7) * 256 + (ja & 127)
            jv_off = lax.shift_right_logical(jv, 7) * 256 + (jv & 127)
            r_off = lax.shift_right_logical(r, 7) * 256 + (r & 127)
            ia_v[sl] = la * cstride + ja_off
            ivp_v[sl] = lv * cstride + jv_off + 128
            ian_v[sl] = la * cstride + r_off + 128
            ivn_v[sl] = lv * cstride + r_off
            pen_v[sl] = jnp.where(la != lv, 0.0, -1e30)
            return 0

        def diff(k, _):
            sl = pl.ds(k * _L, _L)
            da_v[sl] = (van_v[sl] - va_v[sl]) + pen_v[sl]
            dv_v[sl] = (vvn_v[sl] - vvp_v[sl]) + pen_v[sl]
            return 0

        nk = ppw // _L
        lax.fori_loop(0, nk // 2, build, 0)
        h0 = pl.ds(0, half)
        g0 = [
            pltpu.async_copy(pooled_hbm.at[ia_v.at[h0]], va_v.at[h0], sem),
            pltpu.async_copy(pooled_hbm.at[ivp_v.at[h0]], vvp_v.at[h0], sem),
            pltpu.async_copy(pooled_hbm.at[ian_v.at[h0]], van_v.at[h0], sem),
            pltpu.async_copy(pooled_hbm.at[ivn_v.at[h0]], vvn_v.at[h0], sem),
        ]
        lax.fori_loop(nk // 2, nk, build, 0)
        h1 = pl.ds(half, half)
        g1 = [
            pltpu.async_copy(pooled_hbm.at[ia_v.at[h1]], va_v.at[h1], sem2),
            pltpu.async_copy(pooled_hbm.at[ivp_v.at[h1]], vvp_v.at[h1], sem2),
            pltpu.async_copy(pooled_hbm.at[ian_v.at[h1]], van_v.at[h1], sem2),
            pltpu.async_copy(pooled_hbm.at[ivn_v.at[h1]], vvn_v.at[h1], sem2),
        ]
        for cp in g0:
            cp.wait()
        lax.fori_loop(0, nk // 2, diff, 0)
        for cp in g1:
            cp.wait()
        lax.fori_loop(nk // 2, nk, diff, 0)
        out_copies = [
            pltpu.async_copy(da_v, d_hbm.at[pl.ds(base, ppw)], sem),
            pltpu.async_copy(dv_v, d_hbm.at[pl.ds(n + base, ppw)], sem),
        ]
        for cp in out_copies:
            cp.wait()

    return sc_pairs


def kernel(all_prob, audio_idx, vis_idx, target):
    total, t_len, _, c = all_prob.shape
    n = audio_idx.shape[0]
    bn = total - n
    rb = total // 128

    audio_idx = audio_idx.astype(jnp.int32)
    vis_idx = vis_idx.astype(jnp.int32)

    ap_v = (all_prob.reshape(rb, 128, t_len, 2, c)
            .transpose(2, 4, 0, 3, 1)
            .reshape(t_len, c, rb * 2, 128))

    cb = 5
    pooled, labels2d = pl.pallas_call(
        functools.partial(_pool_body, cb),
        grid=(c // cb,),
        in_specs=[
            pl.BlockSpec((t_len, cb, rb * 2, 128), lambda i: (0, i, 0, 0)),
            pl.BlockSpec((c, bn), lambda i: (0, 0)),
        ],
        out_specs=[
            pl.BlockSpec((cb, rb * 2, 128), lambda i: (i, 0, 0)),
            pl.BlockSpec((1, bn), lambda i: (0, 0)),
        ],
        out_shape=[
            jax.ShapeDtypeStruct((c, rb * 2, 128), jnp.float32),
            jax.ShapeDtypeStruct((1, bn), jnp.int32),
        ],
    )(ap_v, target.T)

    ppw = n // _NW
    sc_pairs = _make_sc_pairs(n, bn, c, ppw)
    d_all = sc_pairs(pooled.reshape(c * rb * 256), labels2d.reshape(bn),
                     audio_idx, vis_idx)

    rows = 2 * n // 128
    out = pl.pallas_call(
        _final_body,
        out_shape=jax.ShapeDtypeStruct((1, 1), jnp.float32),
    )(d_all.reshape(rows, 128))
    return out[0, 0]
